# baseline (speedup 1.0000x reference)
import functools

import jax
import jax.numpy as jnp
from jax.experimental import pallas as pl
from jax.experimental.pallas import tpu as pltpu

_K = 8


def _tc_gather(state, q_table):
    B = state.shape[0]
    V, O, A = q_table.shape

    def body(idx_ref, *refs):
        out_ref = refs[-1]
        for k in range(_K):
            out_ref[k] = refs[k][0]

    grid_spec = pltpu.PrefetchScalarGridSpec(
        num_scalar_prefetch=1,
        grid=(B // _K,),
        in_specs=[
            pl.BlockSpec(
                (1, O, A),
                index_map=functools.partial(
                    lambda k, i, idx_ref: (idx_ref[_K * i + k], 0, 0), k
                ),
            )
            for k in range(_K)
        ],
        out_specs=pl.BlockSpec((_K, O, A), lambda i, idx_ref: (i, 0, 0)),
    )

    return pl.pallas_call(
        body,
        grid_spec=grid_spec,
        out_shape=jax.ShapeDtypeStruct((B, O, A), q_table.dtype),
    )(state.astype(jnp.int32), *([q_table] * _K))


def kernel(state, q_table):
    return _tc_gather(state, q_table)


# speedup vs baseline: 1.4280x; 1.4280x over previous
import functools

import jax
import jax.numpy as jnp
from jax import lax
from jax.experimental import pallas as pl
from jax.experimental.pallas import tpu as pltpu
from jax.experimental.pallas import tpu_sc as plsc

_NC = 2
_NS = 16
_NW = _NC * _NS
_CHUNK = 128


def _sc_gather(table, idx, B, D):
    b_per_w = B // _NW
    n_chunks = b_per_w // _CHUNK

    mesh = plsc.VectorSubcoreMesh(core_axis_name="c", subcore_axis_name="s")

    @functools.partial(
        pl.kernel,
        mesh=mesh,
        out_type=jax.ShapeDtypeStruct((B, D), jnp.float32),
        scratch_types=[
            pltpu.VMEM((b_per_w,), jnp.int32),
            pltpu.VMEM((2, _CHUNK, D), jnp.float32),
            pltpu.SemaphoreType.DMA,
        ],
    )
    def gather_kernel(table_hbm, idx_hbm, out_hbm, idx_v, rows_v, sem):
        wid = lax.axis_index("s") * _NC + lax.axis_index("c")
        base = wid * b_per_w
        pltpu.sync_copy(idx_hbm.at[pl.ds(base, b_per_w)], idx_v)

        def start(j):
            return pltpu.async_copy(
                table_hbm.at[idx_v.at[pl.ds(j * _CHUNK, _CHUNK)]],
                rows_v.at[j % 2],
                sem,
            )

        copies = [start(0)]
        for j in range(n_chunks):
            if j + 1 < n_chunks:
                copies.append(start(j + 1))
            copies[j].wait()
            pltpu.sync_copy(
                rows_v.at[j % 2], out_hbm.at[pl.ds(base + j * _CHUNK, _CHUNK)]
            )

    return gather_kernel(table, idx)


def kernel(state, q_table):
    V, O, A = q_table.shape
    D = O * A
    B = state.shape[0]
    table = q_table.reshape(V, D)
    table_pad = jnp.pad(table, ((0, 0), (0, 256 - D)))
    idx = state.astype(jnp.int32)
    out = _sc_gather(table_pad, idx, B, 256)
    return out[:, :D].reshape(B, O, A)


# speedup vs baseline: 5.0887x; 3.5635x over previous
import functools

import jax
import jax.numpy as jnp
from jax import lax
from jax.experimental import pallas as pl
from jax.experimental.pallas import tpu as pltpu
from jax.experimental.pallas import tpu_sc as plsc

_NC = 2
_NS = 16
_NW = _NC * _NS
_CHUNK = 128
_HEAD = 128
_PREP_R = 8192


def _tail_prep(table, V, D):
    tail_w = D - _HEAD

    def body(t_ref, o_ref):
        o_ref[:, :tail_w] = t_ref[:, _HEAD:]

    grid = (V + _PREP_R - 1) // _PREP_R
    return pl.pallas_call(
        body,
        grid=(grid,),
        in_specs=[pl.BlockSpec((_PREP_R, D), lambda i: (i, 0))],
        out_specs=pl.BlockSpec((_PREP_R, _HEAD), lambda i: (i, 0)),
        out_shape=jax.ShapeDtypeStruct((V, _HEAD), jnp.float32),
    )(table)


def _sc_gather(table, tail_t, idx, B, D):
    tail_w = D - _HEAD
    b_per_w = B // _NW
    n_chunks = b_per_w // _CHUNK

    mesh = plsc.VectorSubcoreMesh(core_axis_name="c", subcore_axis_name="s")

    @functools.partial(
        pl.kernel,
        mesh=mesh,
        out_type=(
            jax.ShapeDtypeStruct((B, _HEAD), jnp.float32),
            jax.ShapeDtypeStruct((B, _HEAD), jnp.float32),
        ),
        scratch_types=[
            pltpu.VMEM((b_per_w,), jnp.int32),
            pltpu.VMEM((2, _CHUNK, _HEAD), jnp.float32),
            pltpu.VMEM((2, _CHUNK, _HEAD), jnp.float32),
            pltpu.SemaphoreType.DMA,
        ],
    )
    def gather_kernel(table_hbm, tail_hbm, idx_hbm, outa_hbm, outt_hbm,
                      idx_v, rows_v, tails_v, sem):
        wid = lax.axis_index("s") * _NC + lax.axis_index("c")
        base = wid * b_per_w
        pltpu.sync_copy(idx_hbm.at[pl.ds(base, b_per_w)], idx_v)

        def start(j):
            sl = idx_v.at[pl.ds(j * _CHUNK, _CHUNK)]
            return (
                pltpu.async_copy(
                    table_hbm.at[sl, pl.ds(0, _HEAD)], rows_v.at[j % 2], sem
                ),
                pltpu.async_copy(tail_hbm.at[sl], tails_v.at[j % 2], sem),
            )

        copies = [start(0)]
        for j in range(n_chunks):
            if j + 1 < n_chunks:
                copies.append(start(j + 1))
            copies[j][0].wait()
            copies[j][1].wait()
            rows = pl.ds(base + j * _CHUNK, _CHUNK)
            pltpu.sync_copy(rows_v.at[j % 2], outa_hbm.at[rows])
            pltpu.sync_copy(tails_v.at[j % 2], outt_hbm.at[rows])

    return gather_kernel(table, tail_t, idx)


def kernel(state, q_table):
    V, O, A = q_table.shape
    D = O * A
    B = state.shape[0]
    table = q_table.reshape(V, D)
    idx = state.astype(jnp.int32)
    tail_t = _tail_prep(table, V, D)
    out_head, out_tail = _sc_gather(table, tail_t, idx, B, D)
    out = jnp.concatenate([out_head, out_tail[:, : D - _HEAD]], axis=1)
    return out.reshape(B, O, A)


# speedup vs baseline: 5.1581x; 1.0136x over previous
import functools

import jax
import jax.numpy as jnp
from jax import lax
from jax.experimental import pallas as pl
from jax.experimental.pallas import tpu as pltpu
from jax.experimental.pallas import tpu_sc as plsc

_NC = 2
_NS = 16
_NW = _NC * _NS
_CHUNK = 128
_HEAD = 128
_PREP_R = 4096
_NQ = 4
_QB = 24
_Q = _PREP_R * _QB


def _tail_pack(table, V, D):
    tail_w = D - _HEAD

    def body(*refs):
        o_ref = refs[-1]
        pieces = [refs[k][:, _HEAD:] for k in range(_NQ)]
        o_ref[...] = jnp.concatenate(pieces, axis=1)

    return pl.pallas_call(
        body,
        grid=(_QB,),
        in_specs=[
            pl.BlockSpec((_PREP_R, D), functools.partial(
                lambda k, i: (_QB * k + i, 0), k))
            for k in range(_NQ)
        ],
        out_specs=pl.BlockSpec((_PREP_R, _NQ * tail_w), lambda i: (i, 0)),
        out_shape=jax.ShapeDtypeStruct((_Q, _NQ * tail_w), jnp.float32),
    )(*([table] * _NQ))


def _sc_gather(table, tail_t, idx, idx_t, B, D):
    b_per_w = B // _NW
    n_chunks = b_per_w // _CHUNK

    mesh = plsc.VectorSubcoreMesh(core_axis_name="c", subcore_axis_name="s")

    @functools.partial(
        pl.kernel,
        mesh=mesh,
        out_type=(
            jax.ShapeDtypeStruct((B, _HEAD), jnp.float32),
            jax.ShapeDtypeStruct((B, _HEAD), jnp.float32),
        ),
        scratch_types=[
            pltpu.VMEM((b_per_w,), jnp.int32),
            pltpu.VMEM((b_per_w,), jnp.int32),
            pltpu.VMEM((2, _CHUNK, _HEAD), jnp.float32),
            pltpu.VMEM((2, _CHUNK, _HEAD), jnp.float32),
            pltpu.SemaphoreType.DMA,
        ],
    )
    def gather_kernel(table_hbm, tail_hbm, idx_hbm, idxt_hbm,
                      outa_hbm, outt_hbm,
                      idx_v, idxt_v, rows_v, tails_v, sem):
        wid = lax.axis_index("s") * _NC + lax.axis_index("c")
        base = wid * b_per_w
        pltpu.sync_copy(idx_hbm.at[pl.ds(base, b_per_w)], idx_v)
        pltpu.sync_copy(idxt_hbm.at[pl.ds(base, b_per_w)], idxt_v)

        def start(j):
            sl = pl.ds(j * _CHUNK, _CHUNK)
            return (
                pltpu.async_copy(
                    table_hbm.at[idx_v.at[sl], pl.ds(0, _HEAD)],
                    rows_v.at[j % 2], sem,
                ),
                pltpu.async_copy(
                    tail_hbm.at[idxt_v.at[sl]], tails_v.at[j % 2], sem
                ),
            )

        copies = [start(0)]
        for j in range(n_chunks):
            if j + 1 < n_chunks:
                copies.append(start(j + 1))
            copies[j][0].wait()
            copies[j][1].wait()
            rows = pl.ds(base + j * _CHUNK, _CHUNK)
            pltpu.sync_copy(rows_v.at[j % 2], outa_hbm.at[rows])
            pltpu.sync_copy(tails_v.at[j % 2], outt_hbm.at[rows])

    return gather_kernel(table, tail_t, idx, idx_t)


def kernel(state, q_table):
    V, O, A = q_table.shape
    D = O * A
    B = state.shape[0]
    tail_w = D - _HEAD
    table = q_table.reshape(V, D)
    idx = state.astype(jnp.int32)
    quarter = idx // _Q
    idx_t = idx - quarter * _Q
    tail_t = _tail_pack(table, V, D)
    out_head, out_tail = _sc_gather(table, tail_t, idx, idx_t, B, D)
    packed = out_tail.reshape(B, _NQ, tail_w)
    tails = jnp.take_along_axis(packed, quarter[:, None, None], axis=1)[:, 0]
    out = jnp.concatenate([out_head, tails], axis=1)
    return out.reshape(B, O, A)
